# baseline (speedup 1.0000x reference)
import jax
import jax.numpy as jnp
from jax.experimental import pallas as pl
from jax.experimental.pallas import tpu as pltpu

D_MODEL = 640
INTERMEDIATE_SIZE = 640
EXPERTS_PER_TOKEN = 4
RMS_NORM_EPS = 1e-05
SWIGLU_LIMIT = 7.0
N_EXPERTS = 16
SEQ_LEN = 32
NBUF = 4


def _moe_kernel(xt_ref, nw_ref, gw_ref, gb_ref, m1w_hbm, m1b_ref, m2w_hbm,
                m2b_ref, out_ref, m1buf, m2buf, m1sem, m2sem):
    T, D, I, E, K = SEQ_LEN, D_MODEL, INTERMEDIATE_SIZE, N_EXPERTS, EXPERTS_PER_TOKEN

    def _slab(src, dst, sems, e, slot):
        half = src.shape[1] // 2
        return (pltpu.make_async_copy(src.at[e, :half], dst.at[slot, :half],
                                      sems.at[slot, 0]),
                pltpu.make_async_copy(src.at[e, half:], dst.at[slot, half:],
                                      sems.at[slot, 1]))

    def _start(src, dst, sems, e, slot):
        for c in _slab(src, dst, sems, e, slot):
            c.start()

    def _wait(src, dst, sems, e, slot):
        for c in _slab(src, dst, sems, e, slot):
            c.wait()

    for i in range(NBUF):
        _start(m1w_hbm, m1buf, m1sem, i, i)
        _start(m2w_hbm, m2buf, m2sem, i, i)

    xt = xt_ref[...]
    var = jnp.mean(xt * xt, axis=1, keepdims=True)
    t = xt * jax.lax.rsqrt(var + RMS_NORM_EPS) * nw_ref[...]
    g = jax.lax.dot_general(t, gw_ref[...], (((1,), (1,)), ((), ())),
                            preferred_element_type=jnp.float32)
    g = g + gb_ref[...]
    lane = jax.lax.broadcasted_iota(jnp.int32, (T, E), 1)
    rank = jnp.zeros((T, E), dtype=jnp.int32)
    for j in range(E):
        gj = g[:, j:j + 1]
        rank = rank + (gj > g).astype(jnp.int32)
        rank = rank + ((gj == g) & (j < lane)).astype(jnp.int32)
    sel = rank < K
    gm = jnp.where(sel, g, jnp.float32(-jnp.inf))
    mx = jnp.max(gm, axis=1, keepdims=True)
    ex = jnp.where(sel, jnp.exp(g - mx), 0.0)
    w = ex / jnp.sum(ex, axis=1, keepdims=True)

    acc = xt
    for e in range(E):
        slot = e % NBUF
        _wait(m1w_hbm, m1buf, m1sem, e, slot)
        h = jnp.dot(t, m1buf[slot], preferred_element_type=jnp.float32)
        h = h + m1b_ref[e:e + 1, :]
        h_glu = jnp.minimum(h[:, :I], SWIGLU_LIMIT)
        h_lin = jnp.clip(h[:, I:], -SWIGLU_LIMIT, SWIGLU_LIMIT)
        act = h_glu * jax.nn.sigmoid(1.702 * h_glu) * (h_lin + 1.0)
        _wait(m2w_hbm, m2buf, m2sem, e, slot)
        o = jnp.dot(act, m2buf[slot], preferred_element_type=jnp.float32)
        o = o + m2b_ref[e:e + 1, :]
        acc = acc + w[:, e:e + 1] * o
        nxt = e + NBUF
        if nxt < E:
            _start(m1w_hbm, m1buf, m1sem, nxt, slot)
            _start(m2w_hbm, m2buf, m2sem, nxt, slot)
    out_ref[...] = acc


@jax.jit
def kernel(x, norm_weight, gate_weight, gate_bias, mlp1_weight, mlp1_bias,
           mlp2_weight, mlp2_bias):
    T, D, I, E = SEQ_LEN, D_MODEL, INTERMEDIATE_SIZE, N_EXPERTS
    xt = x.reshape(D, T).T
    out = pl.pallas_call(
        _moe_kernel,
        in_specs=[
            pl.BlockSpec((T, D), lambda: (0, 0)),
            pl.BlockSpec((1, D), lambda: (0, 0)),
            pl.BlockSpec((E, D), lambda: (0, 0)),
            pl.BlockSpec((1, E), lambda: (0, 0)),
            pl.BlockSpec(memory_space=pl.ANY),
            pl.BlockSpec((E, 2 * I), lambda: (0, 0)),
            pl.BlockSpec(memory_space=pl.ANY),
            pl.BlockSpec((E, D), lambda: (0, 0)),
        ],
        out_specs=pl.BlockSpec((T, D), lambda: (0, 0)),
        out_shape=jax.ShapeDtypeStruct((T, D), jnp.float32),
        scratch_shapes=[
            pltpu.VMEM((NBUF, D, 2 * I), jnp.float32),
            pltpu.VMEM((NBUF, I, D), jnp.float32),
            pltpu.SemaphoreType.DMA((NBUF, 2)),
            pltpu.SemaphoreType.DMA((NBUF, 2)),
        ],
    )(xt, norm_weight.reshape(1, D), gate_weight, gate_bias.reshape(1, E),
      mlp1_weight, mlp1_bias, mlp2_weight, mlp2_bias)
    return out.T.reshape(1, D, 1, T)
